# baseline (speedup 1.0000x reference)
import functools

import jax
import jax.numpy as jnp
from jax import lax
from jax.experimental import pallas as pl
from jax.experimental.pallas import tpu as pltpu
from jax.experimental.pallas import tpu_sc as plsc

_NC = 2
_NS = 16
_NW = _NC * _NS
_L = 16


def _make_kernel(n):
    assert n % _L == 0
    nvec = n // _L
    vmain = -(-nvec // _NW)
    vlast = nvec - (_NW - 1) * vmain
    assert 0 < vlast <= vmain
    cmain = vmain * _L
    clast = vlast * _L

    mesh = plsc.VectorSubcoreMesh(core_axis_name="c", subcore_axis_name="s")

    @functools.partial(
        pl.kernel,
        out_type=jax.ShapeDtypeStruct((n,), jnp.float32),
        mesh=mesh,
        compiler_params=pltpu.CompilerParams(needs_layout_passes=False),
        scratch_types=[
            pltpu.VMEM((cmain,), jnp.float32),
            pltpu.VMEM((cmain,), jnp.int32),
            pltpu.VMEM((cmain,), jnp.float32),
            pltpu.VMEM((_L,), jnp.float32),
            pltpu.VMEM((_L,), jnp.float32),
            pltpu.SemaphoreType.DMA,
            pltpu.SemaphoreType.DMA,
        ],
    )
    def rescale(x_hbm, t_hbm, scale_hbm, shift_hbm, out_hbm,
                x_v, t_v, o_v, scale_v, shift_v, sem_in, sem_out):
        wid = lax.axis_index("s") * _NC + lax.axis_index("c")
        base = wid * cmain
        is_last = wid == _NW - 1

        cs = pltpu.async_copy(scale_hbm, scale_v, sem_in)
        cb = pltpu.async_copy(shift_hbm, shift_v, sem_in)
        main_in = [
            pltpu.make_async_copy(x_hbm.at[pl.ds(base, cmain)], x_v, sem_in),
            pltpu.make_async_copy(t_hbm.at[pl.ds(base, cmain)], t_v, sem_in),
        ]
        main_out = pltpu.make_async_copy(
            o_v, out_hbm.at[pl.ds(base, cmain)], sem_out)
        last_in = [
            pltpu.make_async_copy(x_hbm.at[pl.ds(base, clast)],
                                  x_v.at[pl.ds(0, clast)], sem_in),
            pltpu.make_async_copy(t_hbm.at[pl.ds(base, clast)],
                                  t_v.at[pl.ds(0, clast)], sem_in),
        ]
        last_out = pltpu.make_async_copy(
            o_v.at[pl.ds(0, clast)], out_hbm.at[pl.ds(base, clast)], sem_out)

        @pl.when(jnp.logical_not(is_last))
        def _():
            for c in main_in:
                c.start()

        @pl.when(is_last)
        def _():
            for c in last_in:
                c.start()

        cs.wait()
        cb.wait()

        @pl.when(jnp.logical_not(is_last))
        def _():
            for c in main_in:
                c.wait()

        @pl.when(is_last)
        def _():
            for c in last_in:
                c.wait()

        upper = jnp.where(is_last, clast, cmain)

        @plsc.parallel_loop(0, upper, step=_L, unroll=16)
        def _(off):
            idx = t_v[pl.ds(off, _L)]
            xv = x_v[pl.ds(off, _L)]
            s = plsc.load_gather(scale_v, [idx])
            b = plsc.load_gather(shift_v, [idx])
            o_v[pl.ds(off, _L)] = xv * s + b

        @pl.when(jnp.logical_not(is_last))
        def _():
            main_out.start()
            main_out.wait()

        @pl.when(is_last)
        def _():
            last_out.start()
            last_out.wait()

    return rescale


def kernel(scaled_atomic_energy, atom_type, scale, shift):
    n = scaled_atomic_energy.shape[0]
    x = scaled_atomic_energy.reshape(n)
    t = atom_type.astype(jnp.int32)
    out = _make_kernel(n)(x, t, scale, shift)
    return out.reshape(n, 1)
